# baseline (speedup 1.0000x reference)
import functools

import jax
import jax.numpy as jnp
from jax import lax
from jax.experimental import pallas as pl
from jax.experimental.pallas import tpu as pltpu
from jax.experimental.pallas import tpu_sc as plsc

N = 10000
H = 16
NC, NS = 2, 16
CHUNK = 128
NB = 10
CPT0, CPT1 = 30, 50
R = 10112
RPT = R // NS
BM = 400

_mesh = plsc.VectorSubcoreMesh(core_axis_name="c", subcore_axis_name="s")
_sc_params = pltpu.CompilerParams(use_tc_tiling_on_sc=False)


def _sc_degree(dst2d):
    nchunks = dst2d.shape[0]
    assert nchunks == NS * (CPT0 + CPT1) + abs(CPT0 - CPT1)

    @functools.partial(
        pl.kernel,
        mesh=_mesh,
        compiler_params=_sc_params,
        out_type=jax.ShapeDtypeStruct((NC, R, H), jnp.float32),
        scratch_types=[
            pltpu.VMEM((max(CPT0, CPT1), CHUNK), jnp.int32),
            pltpu.VMEM((CHUNK, H), jnp.float32),
            pltpu.VMEM((RPT, H), jnp.float32),
            pltpu.VMEM_SHARED((R, H), jnp.float32),
            pltpu.SemaphoreType.DMA((NB,)),
        ],
    )
    def deg_kernel(dst_hbm, out_hbm, dsts, ones_v, zero_v, acc, ssem):
        cid = lax.axis_index("c")
        sid = lax.axis_index("s")
        cpt = jnp.where(cid == 0, CPT0, CPT1)
        base = sid * (CPT0 + CPT1) + cid * CPT0

        def fill_ones(i, _):
            ones_v[i, :] = jnp.ones((H,), jnp.float32)
            return 0

        lax.fori_loop(0, CHUNK, fill_ones, 0)

        def fill_zero(i, _):
            zero_v[i, :] = jnp.zeros((H,), jnp.float32)
            return 0

        lax.fori_loop(0, RPT, fill_zero, 0)

        pltpu.sync_copy(dst_hbm.at[pl.ds(base, max(CPT0, CPT1))], dsts.at[pl.ds(0, max(CPT0, CPT1))])
        pltpu.sync_copy(zero_v, acc.at[pl.ds(sid * RPT, RPT)])
        plsc.subcore_barrier()

        for b in range(NB):
            pltpu.async_copy(ones_v, acc.at[dsts.at[b]], ssem.at[b],
                             add=True)

        def group(g, _):
            for b in range(NB):
                c = g * NB + b
                pltpu.make_async_copy(ones_v, acc.at[dsts.at[c]],
                                      ssem.at[b]).wait()
                nxt = c + NB

                @pl.when(nxt < cpt)
                def _():
                    pltpu.async_copy(ones_v, acc.at[dsts.at[nxt]],
                                     ssem.at[b], add=True)
            return 0

        lax.fori_loop(0, cpt // NB, group, 0)
        plsc.subcore_barrier()
        pltpu.sync_copy(acc.at[pl.ds(sid * RPT, RPT)],
                        out_hbm.at[cid, pl.ds(sid * RPT, RPT)])

    return deg_kernel(dst2d)


def _sc_scatter(hs, src2d, dst2d):
    nchunks = src2d.shape[0]
    assert nchunks == NS * (CPT0 + CPT1) + abs(CPT0 - CPT1)

    @functools.partial(
        pl.kernel,
        mesh=_mesh,
        compiler_params=_sc_params,
        out_type=jax.ShapeDtypeStruct((NC, R, H), jnp.float32),
        scratch_types=[
            pltpu.VMEM((max(CPT0, CPT1), CHUNK), jnp.int32),
            pltpu.VMEM((max(CPT0, CPT1), CHUNK), jnp.int32),
            pltpu.VMEM((NB, 2, CHUNK, H), jnp.float32),
            pltpu.VMEM((RPT, H), jnp.float32),
            pltpu.VMEM_SHARED((R, H), jnp.float32),
            pltpu.SemaphoreType.DMA((NB, 2)),
        ],
    )
    def scat_kernel(hs_hbm, src_hbm, dst_hbm, out_hbm,
                    srcs, dsts, rows, zero_v, acc, gsem):
        cid = lax.axis_index("c")
        sid = lax.axis_index("s")
        cpt = jnp.where(cid == 0, CPT0, CPT1)
        base = sid * (CPT0 + CPT1) + cid * CPT0

        def fill_zero(i, _):
            zero_v[i, :] = jnp.zeros((H,), jnp.float32)
            return 0

        lax.fori_loop(0, RPT, fill_zero, 0)

        pltpu.sync_copy(src_hbm.at[pl.ds(base, max(CPT0, CPT1))], srcs.at[pl.ds(0, max(CPT0, CPT1))])
        pltpu.sync_copy(dst_hbm.at[pl.ds(base, max(CPT0, CPT1))], dsts.at[pl.ds(0, max(CPT0, CPT1))])
        pltpu.sync_copy(zero_v, acc.at[pl.ds(sid * RPT, RPT)])
        plsc.subcore_barrier()

        for b in range(NB):
            pltpu.async_copy(hs_hbm.at[srcs.at[b]], rows.at[b, 0],
                             gsem.at[b, 0])

        def group(g, _):
            p = lax.rem(g, 2)
            for b in range(NB):
                c = g * NB + b
                pltpu.make_async_copy(hs_hbm.at[srcs.at[c]],
                                      rows.at[b, p], gsem.at[b, p]).wait()

                @pl.when(c + NB < cpt)
                def _():
                    pltpu.async_copy(hs_hbm.at[srcs.at[c + NB]],
                                     rows.at[b, 1 - p], gsem.at[b, 1 - p])

                pltpu.sync_copy(rows.at[b, p], acc.at[dsts.at[c]], add=True)
            return 0

        lax.fori_loop(0, cpt // NB, group, 0)
        plsc.subcore_barrier()
        pltpu.sync_copy(acc.at[pl.ds(sid * RPT, RPT)],
                        out_hbm.at[cid, pl.ds(sid * RPT, RPT)])

    return scat_kernel(hs, src2d, dst2d)


def _dinv_from(degp_ref):
    deg = degp_ref[0, :, 0] + degp_ref[1, :, 0] + 1.0
    return lax.rsqrt(deg)


def _mm1_body(x_ref, w_ref, degp_ref, hs_ref):
    dinv = _dinv_from(degp_ref)
    t = jnp.dot(x_ref[...].astype(jnp.bfloat16),
                w_ref[...].astype(jnp.bfloat16),
                preferred_element_type=jnp.float32)
    hs_ref[...] = t * dinv[:, None]


def _tc_layer1(x, W1, degp):
    grid = N // BM
    return pl.pallas_call(
        _mm1_body,
        grid=(grid,),
        in_specs=[
            pl.BlockSpec((BM, N), lambda i: (i, 0)),
            pl.BlockSpec((N, H), lambda i: (0, 0)),
            pl.BlockSpec((NC, BM, H), lambda i: (0, i, 0)),
        ],
        out_specs=pl.BlockSpec((BM, H), lambda i: (i, 0)),
        out_shape=jax.ShapeDtypeStruct((N, H), jnp.float32),
    )(x, W1, degp)


def _mid_body(s1p_ref, degp_ref, hs1_ref, b1_ref, w2_ref, hs2_ref):
    deg = degp_ref[0, :N, 0] + degp_ref[1, :N, 0] + 1.0
    dinv = lax.rsqrt(deg)
    agg = s1p_ref[0, :N, :] + s1p_ref[1, :N, :] + hs1_ref[...]
    h1 = jnp.maximum(agg * dinv[:, None] + b1_ref[0, :], 0.0)
    t2 = jnp.dot(h1, w2_ref[...], preferred_element_type=jnp.float32)
    hs2_ref[...] = t2 * dinv[:, None]


def _tc_mid(s1p, degp, hs1, b1, W2):
    return pl.pallas_call(
        _mid_body,
        in_specs=[
            pl.BlockSpec((NC, R, H), lambda: (0, 0, 0)),
            pl.BlockSpec((NC, R, H), lambda: (0, 0, 0)),
            pl.BlockSpec((N, H), lambda: (0, 0)),
            pl.BlockSpec((1, H), lambda: (0, 0)),
            pl.BlockSpec((H, H), lambda: (0, 0)),
        ],
        out_specs=pl.BlockSpec((N, H), lambda: (0, 0)),
        out_shape=jax.ShapeDtypeStruct((N, H), jnp.float32),
    )(s1p, degp, hs1, b1.reshape(1, H), W2)


def _out_body(s2p_ref, degp_ref, hs2_ref, b2_ref, wout_ref, bout_ref, o_ref):
    dinv = _dinv_from(degp_ref)
    agg = s2p_ref[0] + s2p_ref[1] + hs2_ref[...]
    h2 = jnp.maximum(agg * dinv[:, None] + b2_ref[0, :], 0.0)
    o_ref[...] = jnp.dot(h2.astype(jnp.bfloat16),
                         wout_ref[...].astype(jnp.bfloat16),
                         preferred_element_type=jnp.float32) + bout_ref[0, :]


def _tc_out(s2p, degp, hs2, b2, Wout, bout):
    grid = N // BM
    return pl.pallas_call(
        _out_body,
        grid=(grid,),
        in_specs=[
            pl.BlockSpec((NC, BM, H), lambda i: (0, i, 0)),
            pl.BlockSpec((NC, BM, H), lambda i: (0, i, 0)),
            pl.BlockSpec((BM, H), lambda i: (i, 0)),
            pl.BlockSpec((1, H), lambda i: (0, 0)),
            pl.BlockSpec((H, N), lambda i: (0, 0)),
            pl.BlockSpec((1, N), lambda i: (0, 0)),
        ],
        out_specs=pl.BlockSpec((BM, N), lambda i: (i, 0)),
        out_shape=jax.ShapeDtypeStruct((N, N), jnp.float32),
    )(s2p, degp, hs2, b2.reshape(1, H), Wout, bout.reshape(1, N))


def kernel(x, edge_index, W1, b1, W2, b2, Wout, bout):
    E = edge_index.shape[1]
    nch = NS * (CPT0 + CPT1) + abs(CPT0 - CPT1)
    EP = nch * CHUNK
    pad = EP - E
    lane = jnp.arange(pad, dtype=jnp.int32) % 16
    src = jnp.concatenate([edge_index[0], lane]).reshape(nch, CHUNK)
    dst = jnp.concatenate([edge_index[1], N + lane]).reshape(nch, CHUNK)

    degp = _sc_degree(dst)
    hs1 = _tc_layer1(x, W1, degp)
    s1p = _sc_scatter(hs1, src, dst)
    hs2 = _tc_mid(s1p, degp, hs1, b1, W2)
    s2p = _sc_scatter(hs2, src, dst)
    return _tc_out(s2p, degp, hs2, b2, Wout, bout)


# speedup vs baseline: 1.0033x; 1.0033x over previous
import functools

import jax
import jax.numpy as jnp
from jax import lax
from jax.experimental import pallas as pl
from jax.experimental.pallas import tpu as pltpu
from jax.experimental.pallas import tpu_sc as plsc

N = 10000
H = 16
NC, NS = 2, 16
CHUNK = 128
NB = 10
CPT0, CPT1 = 50, 30
R = 10112
RPT = R // NS
BM = 400

_mesh = plsc.VectorSubcoreMesh(core_axis_name="c", subcore_axis_name="s")
_sc_params = pltpu.CompilerParams(use_tc_tiling_on_sc=False)


def _sc_degree(dst2d):
    nchunks = dst2d.shape[0]
    assert nchunks == NS * (CPT0 + CPT1) + abs(CPT0 - CPT1)

    @functools.partial(
        pl.kernel,
        mesh=_mesh,
        compiler_params=_sc_params,
        out_type=jax.ShapeDtypeStruct((NC, R, H), jnp.float32),
        scratch_types=[
            pltpu.VMEM((max(CPT0, CPT1), CHUNK), jnp.int32),
            pltpu.VMEM((CHUNK, H), jnp.float32),
            pltpu.VMEM((RPT, H), jnp.float32),
            pltpu.VMEM_SHARED((R, H), jnp.float32),
            pltpu.SemaphoreType.DMA((NB,)),
        ],
    )
    def deg_kernel(dst_hbm, out_hbm, dsts, ones_v, zero_v, acc, ssem):
        cid = lax.axis_index("c")
        sid = lax.axis_index("s")
        cpt = jnp.where(cid == 0, CPT0, CPT1)
        base = sid * (CPT0 + CPT1) + cid * CPT0

        def fill_ones(i, _):
            ones_v[i, :] = jnp.ones((H,), jnp.float32)
            return 0

        lax.fori_loop(0, CHUNK, fill_ones, 0)

        def fill_zero(i, _):
            zero_v[i, :] = jnp.zeros((H,), jnp.float32)
            return 0

        lax.fori_loop(0, RPT, fill_zero, 0)

        pltpu.sync_copy(dst_hbm.at[pl.ds(base, max(CPT0, CPT1))], dsts.at[pl.ds(0, max(CPT0, CPT1))])
        pltpu.sync_copy(zero_v, acc.at[pl.ds(sid * RPT, RPT)])
        plsc.subcore_barrier()

        for b in range(NB):
            pltpu.async_copy(ones_v, acc.at[dsts.at[b]], ssem.at[b],
                             add=True)

        def group(g, _):
            for b in range(NB):
                c = g * NB + b
                pltpu.make_async_copy(ones_v, acc.at[dsts.at[c]],
                                      ssem.at[b]).wait()
                nxt = c + NB

                @pl.when(nxt < cpt)
                def _():
                    pltpu.async_copy(ones_v, acc.at[dsts.at[nxt]],
                                     ssem.at[b], add=True)
            return 0

        lax.fori_loop(0, cpt // NB, group, 0)
        plsc.subcore_barrier()
        pltpu.sync_copy(acc.at[pl.ds(sid * RPT, RPT)],
                        out_hbm.at[cid, pl.ds(sid * RPT, RPT)])

    return deg_kernel(dst2d)


def _sc_scatter(hs, src2d, dst2d):
    nchunks = src2d.shape[0]
    assert nchunks == NS * (CPT0 + CPT1) + abs(CPT0 - CPT1)

    @functools.partial(
        pl.kernel,
        mesh=_mesh,
        compiler_params=_sc_params,
        out_type=jax.ShapeDtypeStruct((NC, R, H), jnp.float32),
        scratch_types=[
            pltpu.VMEM((max(CPT0, CPT1), CHUNK), jnp.int32),
            pltpu.VMEM((max(CPT0, CPT1), CHUNK), jnp.int32),
            pltpu.VMEM((NB, 2, CHUNK, H), jnp.float32),
            pltpu.VMEM((RPT, H), jnp.float32),
            pltpu.VMEM_SHARED((R, H), jnp.float32),
            pltpu.SemaphoreType.DMA((NB, 2)),
        ],
    )
    def scat_kernel(hs_hbm, src_hbm, dst_hbm, out_hbm,
                    srcs, dsts, rows, zero_v, acc, gsem):
        cid = lax.axis_index("c")
        sid = lax.axis_index("s")
        cpt = jnp.where(cid == 0, CPT0, CPT1)
        base = sid * (CPT0 + CPT1) + cid * CPT0

        def fill_zero(i, _):
            zero_v[i, :] = jnp.zeros((H,), jnp.float32)
            return 0

        lax.fori_loop(0, RPT, fill_zero, 0)

        pltpu.sync_copy(src_hbm.at[pl.ds(base, max(CPT0, CPT1))], srcs.at[pl.ds(0, max(CPT0, CPT1))])
        pltpu.sync_copy(dst_hbm.at[pl.ds(base, max(CPT0, CPT1))], dsts.at[pl.ds(0, max(CPT0, CPT1))])
        pltpu.sync_copy(zero_v, acc.at[pl.ds(sid * RPT, RPT)])
        plsc.subcore_barrier()

        for b in range(NB):
            pltpu.async_copy(hs_hbm.at[srcs.at[b]], rows.at[b, 0],
                             gsem.at[b, 0])

        def group(g, _):
            p = lax.rem(g, 2)
            for b in range(NB):
                c = g * NB + b
                pltpu.make_async_copy(hs_hbm.at[srcs.at[c]],
                                      rows.at[b, p], gsem.at[b, p]).wait()

                @pl.when(c + NB < cpt)
                def _():
                    pltpu.async_copy(hs_hbm.at[srcs.at[c + NB]],
                                     rows.at[b, 1 - p], gsem.at[b, 1 - p])

                pltpu.sync_copy(rows.at[b, p], acc.at[dsts.at[c]], add=True)
            return 0

        lax.fori_loop(0, cpt // NB, group, 0)
        plsc.subcore_barrier()
        pltpu.sync_copy(acc.at[pl.ds(sid * RPT, RPT)],
                        out_hbm.at[cid, pl.ds(sid * RPT, RPT)])

    return scat_kernel(hs, src2d, dst2d)


def _dinv_from(degp_ref):
    deg = degp_ref[0, :, 0] + degp_ref[1, :, 0] + 1.0
    return lax.rsqrt(deg)


def _mm1_body(x_ref, w_ref, degp_ref, hs_ref):
    dinv = _dinv_from(degp_ref)
    t = jnp.dot(x_ref[...].astype(jnp.bfloat16),
                w_ref[...].astype(jnp.bfloat16),
                preferred_element_type=jnp.float32)
    hs_ref[...] = t * dinv[:, None]


def _tc_layer1(x, W1, degp):
    grid = N // BM
    return pl.pallas_call(
        _mm1_body,
        grid=(grid,),
        in_specs=[
            pl.BlockSpec((BM, N), lambda i: (i, 0)),
            pl.BlockSpec((N, H), lambda i: (0, 0)),
            pl.BlockSpec((NC, BM, H), lambda i: (0, i, 0)),
        ],
        out_specs=pl.BlockSpec((BM, H), lambda i: (i, 0)),
        out_shape=jax.ShapeDtypeStruct((N, H), jnp.float32),
    )(x, W1, degp)


def _mid_body(s1p_ref, degp_ref, hs1_ref, b1_ref, w2_ref, hs2_ref):
    deg = degp_ref[0, :N, 0] + degp_ref[1, :N, 0] + 1.0
    dinv = lax.rsqrt(deg)
    agg = s1p_ref[0, :N, :] + s1p_ref[1, :N, :] + hs1_ref[...]
    h1 = jnp.maximum(agg * dinv[:, None] + b1_ref[0, :], 0.0)
    t2 = jnp.dot(h1, w2_ref[...], preferred_element_type=jnp.float32)
    hs2_ref[...] = t2 * dinv[:, None]


def _tc_mid(s1p, degp, hs1, b1, W2):
    return pl.pallas_call(
        _mid_body,
        in_specs=[
            pl.BlockSpec((NC, R, H), lambda: (0, 0, 0)),
            pl.BlockSpec((NC, R, H), lambda: (0, 0, 0)),
            pl.BlockSpec((N, H), lambda: (0, 0)),
            pl.BlockSpec((1, H), lambda: (0, 0)),
            pl.BlockSpec((H, H), lambda: (0, 0)),
        ],
        out_specs=pl.BlockSpec((N, H), lambda: (0, 0)),
        out_shape=jax.ShapeDtypeStruct((N, H), jnp.float32),
    )(s1p, degp, hs1, b1.reshape(1, H), W2)


def _out_body(s2p_ref, degp_ref, hs2_ref, b2_ref, wout_ref, bout_ref, o_ref):
    dinv = _dinv_from(degp_ref)
    agg = s2p_ref[0] + s2p_ref[1] + hs2_ref[...]
    h2 = jnp.maximum(agg * dinv[:, None] + b2_ref[0, :], 0.0)
    o_ref[...] = jnp.dot(h2.astype(jnp.bfloat16),
                         wout_ref[...].astype(jnp.bfloat16),
                         preferred_element_type=jnp.float32) + bout_ref[0, :]


def _tc_out(s2p, degp, hs2, b2, Wout, bout):
    grid = N // BM
    return pl.pallas_call(
        _out_body,
        grid=(grid,),
        in_specs=[
            pl.BlockSpec((NC, BM, H), lambda i: (0, i, 0)),
            pl.BlockSpec((NC, BM, H), lambda i: (0, i, 0)),
            pl.BlockSpec((BM, H), lambda i: (i, 0)),
            pl.BlockSpec((1, H), lambda i: (0, 0)),
            pl.BlockSpec((H, N), lambda i: (0, 0)),
            pl.BlockSpec((1, N), lambda i: (0, 0)),
        ],
        out_specs=pl.BlockSpec((BM, N), lambda i: (i, 0)),
        out_shape=jax.ShapeDtypeStruct((N, N), jnp.float32),
    )(s2p, degp, hs2, b2.reshape(1, H), Wout, bout.reshape(1, N))


def kernel(x, edge_index, W1, b1, W2, b2, Wout, bout):
    E = edge_index.shape[1]
    nch = NS * (CPT0 + CPT1) + abs(CPT0 - CPT1)
    EP = nch * CHUNK
    pad = EP - E
    lane = jnp.arange(pad, dtype=jnp.int32) % 16
    src = jnp.concatenate([edge_index[0], lane]).reshape(nch, CHUNK)
    dst = jnp.concatenate([edge_index[1], N + lane]).reshape(nch, CHUNK)

    degp = _sc_degree(dst)
    hs1 = _tc_layer1(x, W1, degp)
    s1p = _sc_scatter(hs1, src, dst)
    hs2 = _tc_mid(s1p, degp, hs1, b1, W2)
    s2p = _sc_scatter(hs2, src, dst)
    return _tc_out(s2p, degp, hs2, b2, Wout, bout)


# speedup vs baseline: 1.0255x; 1.0221x over previous
import functools

import jax
import jax.numpy as jnp
from jax import lax
from jax.experimental import pallas as pl
from jax.experimental.pallas import tpu as pltpu
from jax.experimental.pallas import tpu_sc as plsc

N = 10000
H = 16
NC, NS = 2, 16
CHUNK = 128
NB = 10
CPT0, CPT1 = 50, 30
R = 10112
RPT = R // NS
BM = 400

_mesh = plsc.VectorSubcoreMesh(core_axis_name="c", subcore_axis_name="s")
_sc_params = pltpu.CompilerParams(use_tc_tiling_on_sc=False)


def _sc_degree(dst2d):
    nchunks = dst2d.shape[0]
    assert nchunks == NS * (CPT0 + CPT1) + abs(CPT0 - CPT1)

    @functools.partial(
        pl.kernel,
        mesh=_mesh,
        compiler_params=_sc_params,
        out_type=jax.ShapeDtypeStruct((NC, R, H), jnp.float32),
        scratch_types=[
            pltpu.VMEM((max(CPT0, CPT1), CHUNK), jnp.int32),
            pltpu.VMEM((CHUNK, H), jnp.float32),
            pltpu.VMEM((RPT, H), jnp.float32),
            pltpu.VMEM_SHARED((R, H), jnp.float32),
            pltpu.SemaphoreType.DMA((NB,)),
        ],
    )
    def deg_kernel(dst_hbm, out_hbm, dsts, ones_v, zero_v, acc, ssem):
        cid = lax.axis_index("c")
        sid = lax.axis_index("s")
        cpt = jnp.where(cid == 0, CPT0, CPT1)
        base = sid * (CPT0 + CPT1) + cid * CPT0

        def fill_ones(i, _):
            ones_v[i, :] = jnp.ones((H,), jnp.float32)
            return 0

        lax.fori_loop(0, CHUNK, fill_ones, 0)

        def fill_zero(i, _):
            zero_v[i, :] = jnp.zeros((H,), jnp.float32)
            return 0

        lax.fori_loop(0, RPT, fill_zero, 0)

        pltpu.sync_copy(dst_hbm.at[pl.ds(base, max(CPT0, CPT1))], dsts.at[pl.ds(0, max(CPT0, CPT1))])
        pltpu.sync_copy(zero_v, acc.at[pl.ds(sid * RPT, RPT)])
        plsc.subcore_barrier()

        for b in range(NB):
            pltpu.async_copy(ones_v, acc.at[dsts.at[b]], ssem.at[b],
                             add=True)

        def group(g, _):
            for b in range(NB):
                c = g * NB + b
                pltpu.make_async_copy(ones_v, acc.at[dsts.at[c]],
                                      ssem.at[b]).wait()
                nxt = c + NB

                @pl.when(nxt < cpt)
                def _():
                    pltpu.async_copy(ones_v, acc.at[dsts.at[nxt]],
                                     ssem.at[b], add=True)
            return 0

        lax.fori_loop(0, cpt // NB, group, 0)
        plsc.subcore_barrier()
        pltpu.sync_copy(acc.at[pl.ds(sid * RPT, RPT)],
                        out_hbm.at[cid, pl.ds(sid * RPT, RPT)])

    return deg_kernel(dst2d)


def _sc_scatter(hs, src2d, dst2d):
    nchunks = src2d.shape[0]
    assert nchunks == NS * (CPT0 + CPT1) + abs(CPT0 - CPT1)

    @functools.partial(
        pl.kernel,
        mesh=_mesh,
        compiler_params=_sc_params,
        out_type=jax.ShapeDtypeStruct((NC, R, H), jnp.float32),
        scratch_types=[
            pltpu.VMEM((max(CPT0, CPT1), CHUNK), jnp.int32),
            pltpu.VMEM((max(CPT0, CPT1), CHUNK), jnp.int32),
            pltpu.VMEM((NB, 2, CHUNK, H), jnp.float32),
            pltpu.VMEM((RPT, H), jnp.float32),
            pltpu.VMEM_SHARED((R, H), jnp.float32),
            pltpu.VMEM_SHARED((N, H), jnp.float32),
            pltpu.SemaphoreType.DMA((NB, 2)),
        ],
    )
    def scat_kernel(hs_hbm, src_hbm, dst_hbm, out_hbm,
                    srcs, dsts, rows, zero_v, acc, table, gsem):
        cid = lax.axis_index("c")
        sid = lax.axis_index("s")
        cpt = jnp.where(cid == 0, CPT0, CPT1)
        base = sid * (CPT0 + CPT1) + cid * CPT0

        def fill_zero(i, _):
            zero_v[i, :] = jnp.zeros((H,), jnp.float32)
            return 0

        lax.fori_loop(0, RPT, fill_zero, 0)

        pltpu.sync_copy(src_hbm.at[pl.ds(base, max(CPT0, CPT1))], srcs.at[pl.ds(0, max(CPT0, CPT1))])
        pltpu.sync_copy(dst_hbm.at[pl.ds(base, max(CPT0, CPT1))], dsts.at[pl.ds(0, max(CPT0, CPT1))])
        pltpu.sync_copy(zero_v, acc.at[pl.ds(sid * RPT, RPT)])
        pltpu.sync_copy(hs_hbm.at[pl.ds(sid * (N // NS), N // NS)],
                        table.at[pl.ds(sid * (N // NS), N // NS)])
        plsc.subcore_barrier()

        for b in range(NB):
            pltpu.async_copy(table.at[srcs.at[b]], rows.at[b, 0],
                             gsem.at[b, 0])

        def group(g, _):
            p = lax.rem(g, 2)
            for b in range(NB):
                c = g * NB + b
                pltpu.make_async_copy(table.at[srcs.at[c]],
                                      rows.at[b, p], gsem.at[b, p]).wait()

                @pl.when(c + NB < cpt)
                def _():
                    pltpu.async_copy(table.at[srcs.at[c + NB]],
                                     rows.at[b, 1 - p], gsem.at[b, 1 - p])

                pltpu.sync_copy(rows.at[b, p], acc.at[dsts.at[c]], add=True)
            return 0

        lax.fori_loop(0, cpt // NB, group, 0)
        plsc.subcore_barrier()
        pltpu.sync_copy(acc.at[pl.ds(sid * RPT, RPT)],
                        out_hbm.at[cid, pl.ds(sid * RPT, RPT)])

    return scat_kernel(hs, src2d, dst2d)


def _dinv_from(degp_ref):
    deg = degp_ref[0, :, 0] + degp_ref[1, :, 0] + 1.0
    return lax.rsqrt(deg)


def _mm1_body(x_ref, w_ref, degp_ref, hs_ref):
    dinv = _dinv_from(degp_ref)
    t = jnp.dot(x_ref[...].astype(jnp.bfloat16),
                w_ref[...].astype(jnp.bfloat16),
                preferred_element_type=jnp.float32)
    hs_ref[...] = t * dinv[:, None]


def _tc_layer1(x, W1, degp):
    grid = N // BM
    return pl.pallas_call(
        _mm1_body,
        grid=(grid,),
        in_specs=[
            pl.BlockSpec((BM, N), lambda i: (i, 0)),
            pl.BlockSpec((N, H), lambda i: (0, 0)),
            pl.BlockSpec((NC, BM, H), lambda i: (0, i, 0)),
        ],
        out_specs=pl.BlockSpec((BM, H), lambda i: (i, 0)),
        out_shape=jax.ShapeDtypeStruct((N, H), jnp.float32),
    )(x, W1, degp)


def _mid_body(s1p_ref, degp_ref, hs1_ref, b1_ref, w2_ref, hs2_ref):
    deg = degp_ref[0, :N, 0] + degp_ref[1, :N, 0] + 1.0
    dinv = lax.rsqrt(deg)
    agg = s1p_ref[0, :N, :] + s1p_ref[1, :N, :] + hs1_ref[...]
    h1 = jnp.maximum(agg * dinv[:, None] + b1_ref[0, :], 0.0)
    t2 = jnp.dot(h1, w2_ref[...], preferred_element_type=jnp.float32)
    hs2_ref[...] = t2 * dinv[:, None]


def _tc_mid(s1p, degp, hs1, b1, W2):
    return pl.pallas_call(
        _mid_body,
        in_specs=[
            pl.BlockSpec((NC, R, H), lambda: (0, 0, 0)),
            pl.BlockSpec((NC, R, H), lambda: (0, 0, 0)),
            pl.BlockSpec((N, H), lambda: (0, 0)),
            pl.BlockSpec((1, H), lambda: (0, 0)),
            pl.BlockSpec((H, H), lambda: (0, 0)),
        ],
        out_specs=pl.BlockSpec((N, H), lambda: (0, 0)),
        out_shape=jax.ShapeDtypeStruct((N, H), jnp.float32),
    )(s1p, degp, hs1, b1.reshape(1, H), W2)


def _out_body(s2p_ref, degp_ref, hs2_ref, b2_ref, wout_ref, bout_ref, o_ref):
    dinv = _dinv_from(degp_ref)
    agg = s2p_ref[0] + s2p_ref[1] + hs2_ref[...]
    h2 = jnp.maximum(agg * dinv[:, None] + b2_ref[0, :], 0.0)
    o_ref[...] = jnp.dot(h2.astype(jnp.bfloat16),
                         wout_ref[...].astype(jnp.bfloat16),
                         preferred_element_type=jnp.float32) + bout_ref[0, :]


def _tc_out(s2p, degp, hs2, b2, Wout, bout):
    grid = N // BM
    return pl.pallas_call(
        _out_body,
        grid=(grid,),
        in_specs=[
            pl.BlockSpec((NC, BM, H), lambda i: (0, i, 0)),
            pl.BlockSpec((NC, BM, H), lambda i: (0, i, 0)),
            pl.BlockSpec((BM, H), lambda i: (i, 0)),
            pl.BlockSpec((1, H), lambda i: (0, 0)),
            pl.BlockSpec((H, N), lambda i: (0, 0)),
            pl.BlockSpec((1, N), lambda i: (0, 0)),
        ],
        out_specs=pl.BlockSpec((BM, N), lambda i: (i, 0)),
        out_shape=jax.ShapeDtypeStruct((N, N), jnp.float32),
    )(s2p, degp, hs2, b2.reshape(1, H), Wout, bout.reshape(1, N))


def kernel(x, edge_index, W1, b1, W2, b2, Wout, bout):
    E = edge_index.shape[1]
    nch = NS * (CPT0 + CPT1) + abs(CPT0 - CPT1)
    EP = nch * CHUNK
    pad = EP - E
    lane = jnp.arange(pad, dtype=jnp.int32) % 16
    src = jnp.concatenate([edge_index[0], lane]).reshape(nch, CHUNK)
    dst = jnp.concatenate([edge_index[1], N + lane]).reshape(nch, CHUNK)

    degp = _sc_degree(dst)
    hs1 = _tc_layer1(x, W1, degp)
    s1p = _sc_scatter(hs1, src, dst)
    hs2 = _tc_mid(s1p, degp, hs1, b1, W2)
    s2p = _sc_scatter(hs2, src, dst)
    return _tc_out(s2p, degp, hs2, b2, Wout, bout)
